# Initial kernel scaffold; baseline (speedup 1.0000x reference)
import jax
import jax.numpy as jnp
from jax.experimental import pallas as pl


def kernel(x, table):
    raise NotImplementedError("write your pallas kernel here")


# speedup vs baseline: 1.5216x; 1.5216x over previous
import functools

import jax
import jax.numpy as jnp
from jax import lax
from jax.experimental import pallas as pl
from jax.experimental.pallas import tpu as pltpu
from jax.experimental.pallas import tpu_sc as plsc

NUM_CATEGORIES = 1000000
EMBEDDING_DIM = 32
BATCH = 16384
FIELDS = 26

NC = 2
NS = 16
NW = NC * NS

N_LOOKUPS = BATCH * FIELDS
PER_W = N_LOOKUPS // NW
CHUNK = 128
NCHUNK = PER_W // CHUNK
NBUF = 2


def _emb_body(x_hbm, table_hbm, out_hbm, idx_v, rows_v, sem_g, sem_s):
    wid = lax.axis_index("s") * NC + lax.axis_index("c")
    base = wid * PER_W

    pltpu.sync_copy(x_hbm.at[wid], idx_v)

    def gather_start(j, slot):
        pltpu.async_copy(
            table_hbm.at[idx_v.at[j]], rows_v.at[slot], sem_g.at[slot]
        )

    def gather_wait(j, slot):
        pltpu.make_async_copy(
            table_hbm.at[idx_v.at[j]], rows_v.at[slot], sem_g.at[slot]
        ).wait()

    def store_start(j, slot):
        pltpu.async_copy(
            rows_v.at[slot], out_hbm.at[pl.ds(base + j * CHUNK, CHUNK)],
            sem_s.at[slot],
        )

    def store_wait(j, slot):
        pltpu.make_async_copy(
            rows_v.at[slot], out_hbm.at[pl.ds(base + j * CHUNK, CHUNK)],
            sem_s.at[slot],
        ).wait()

    for b in range(NBUF):
        gather_start(b, b)

    def loop_body(c, carry):
        j0 = c * NBUF
        for b in range(NBUF):
            j = j0 + b
            gather_wait(j, b)
            store_start(j, b)
            @pl.when(j + NBUF < NCHUNK)
            def _refill(j=j, b=b):
                store_wait(j, b)
                gather_start(j + NBUF, b)
        return carry

    lax.fori_loop(0, NCHUNK // NBUF, loop_body, 0)

    for b in range(NBUF):
        store_wait(NCHUNK - NBUF + b, b)


def _embedding_lookup(x_w, table):
    mesh = plsc.VectorSubcoreMesh(core_axis_name="c", subcore_axis_name="s")
    f = pl.kernel(
        _emb_body,
        out_type=jax.ShapeDtypeStruct((N_LOOKUPS, EMBEDDING_DIM), jnp.float32),
        mesh=mesh,
        scratch_types=[
            pltpu.VMEM((NCHUNK, CHUNK), jnp.int32),
            pltpu.VMEM((NBUF, CHUNK, EMBEDDING_DIM), jnp.float32),
            pltpu.SemaphoreType.DMA((NBUF,)),
            pltpu.SemaphoreType.DMA((NBUF,)),
        ],
        compiler_params=pltpu.CompilerParams(use_tc_tiling_on_sc=False),
    )
    return f(x_w, table)


def kernel(x, table):
    x_flat = x.reshape(-1).astype(jnp.int32)
    x_w = x_flat.reshape(NW, NCHUNK, CHUNK)
    out = _embedding_lookup(x_w, table)
    return out.reshape(x.shape + (EMBEDDING_DIM,))


# speedup vs baseline: 1.5767x; 1.0363x over previous
import functools

import jax
import jax.numpy as jnp
from jax import lax
from jax.experimental import pallas as pl
from jax.experimental.pallas import tpu as pltpu
from jax.experimental.pallas import tpu_sc as plsc

NUM_CATEGORIES = 1000000
EMBEDDING_DIM = 32
BATCH = 16384
FIELDS = 26

NC = 2
NS = 16
NW = NC * NS

N_LOOKUPS = BATCH * FIELDS
PER_W = N_LOOKUPS // NW
CHUNK = 128
NCHUNK = PER_W // CHUNK
NBUF = 8
LOOKAHEAD = 4


def _emb_body(x_hbm, table_hbm, out_hbm, idx_v, rows_v, sem_g, sem_s):
    wid = lax.axis_index("s") * NC + lax.axis_index("c")
    base = wid * PER_W

    pltpu.sync_copy(x_hbm.at[wid], idx_v)

    def gather_start(j, slot):
        pltpu.async_copy(
            table_hbm.at[idx_v.at[j]], rows_v.at[slot], sem_g.at[slot]
        )

    def gather_wait(j, slot):
        pltpu.make_async_copy(
            table_hbm.at[idx_v.at[j]], rows_v.at[slot], sem_g.at[slot]
        ).wait()

    def store_start(j, slot):
        pltpu.async_copy(
            rows_v.at[slot], out_hbm.at[pl.ds(base + j * CHUNK, CHUNK)],
            sem_s.at[slot],
        )

    def store_wait(j, slot):
        pltpu.make_async_copy(
            rows_v.at[slot], out_hbm.at[pl.ds(base + j * CHUNK, CHUNK)],
            sem_s.at[slot],
        ).wait()

    for b in range(NBUF):
        gather_start(b, b)

    def loop_body(j, carry):
        m = j + LOOKAHEAD

        @pl.when(jnp.logical_and(m >= NBUF, m < NCHUNK))
        def _refill():
            slot = lax.rem(m, NBUF)
            store_wait(m - NBUF, slot)
            gather_start(m, slot)

        slot = lax.rem(j, NBUF)
        gather_wait(j, slot)
        store_start(j, slot)
        return carry

    lax.fori_loop(0, NCHUNK, loop_body, 0)

    for b in range(NBUF):
        j = NCHUNK - NBUF + b
        store_wait(j, j % NBUF)


def _embedding_lookup(x_w, table):
    mesh = plsc.VectorSubcoreMesh(core_axis_name="c", subcore_axis_name="s")
    f = pl.kernel(
        _emb_body,
        out_type=jax.ShapeDtypeStruct((N_LOOKUPS, EMBEDDING_DIM), jnp.float32),
        mesh=mesh,
        scratch_types=[
            pltpu.VMEM((NCHUNK, CHUNK), jnp.int32),
            pltpu.VMEM((NBUF, CHUNK, EMBEDDING_DIM), jnp.float32),
            pltpu.SemaphoreType.DMA((NBUF,)),
            pltpu.SemaphoreType.DMA((NBUF,)),
        ],
        compiler_params=pltpu.CompilerParams(use_tc_tiling_on_sc=False),
    )
    return f(x_w, table)


def kernel(x, table):
    x_flat = x.reshape(-1).astype(jnp.int32)
    x_w = x_flat.reshape(NW, NCHUNK, CHUNK)
    out = _embedding_lookup(x_w, table)
    return out.reshape(x.shape + (EMBEDDING_DIM,))


# speedup vs baseline: 1.5774x; 1.0004x over previous
import functools

import jax
import jax.numpy as jnp
from jax import lax
from jax.experimental import pallas as pl
from jax.experimental.pallas import tpu as pltpu
from jax.experimental.pallas import tpu_sc as plsc

NUM_CATEGORIES = 1000000
EMBEDDING_DIM = 32
BATCH = 16384
FIELDS = 26

NC = 2
NS = 16
NW = NC * NS

N_LOOKUPS = BATCH * FIELDS
PER_W = N_LOOKUPS // NW
CHUNK = 256
NCHUNK = PER_W // CHUNK
NBUF = 8
LOOKAHEAD = 4


def _emb_body(x_hbm, table_hbm, out_hbm, idx_v, rows_v, sem_g, sem_s):
    wid = lax.axis_index("s") * NC + lax.axis_index("c")
    base = wid * PER_W

    pltpu.sync_copy(x_hbm.at[wid], idx_v)

    def gather_start(j, slot):
        pltpu.async_copy(
            table_hbm.at[idx_v.at[j]], rows_v.at[slot], sem_g.at[slot]
        )

    def gather_wait(j, slot):
        pltpu.make_async_copy(
            table_hbm.at[idx_v.at[j]], rows_v.at[slot], sem_g.at[slot]
        ).wait()

    def store_start(j, slot):
        pltpu.async_copy(
            rows_v.at[slot], out_hbm.at[pl.ds(base + j * CHUNK, CHUNK)],
            sem_s.at[slot],
        )

    def store_wait(j, slot):
        pltpu.make_async_copy(
            rows_v.at[slot], out_hbm.at[pl.ds(base + j * CHUNK, CHUNK)],
            sem_s.at[slot],
        ).wait()

    for b in range(NBUF):
        gather_start(b, b)

    def loop_body(j, carry):
        m = j + LOOKAHEAD

        @pl.when(jnp.logical_and(m >= NBUF, m < NCHUNK))
        def _refill():
            slot = lax.rem(m, NBUF)
            store_wait(m - NBUF, slot)
            gather_start(m, slot)

        slot = lax.rem(j, NBUF)
        gather_wait(j, slot)
        store_start(j, slot)
        return carry

    lax.fori_loop(0, NCHUNK, loop_body, 0)

    for b in range(NBUF):
        j = NCHUNK - NBUF + b
        store_wait(j, j % NBUF)


def _embedding_lookup(x_w, table):
    mesh = plsc.VectorSubcoreMesh(core_axis_name="c", subcore_axis_name="s")
    f = pl.kernel(
        _emb_body,
        out_type=jax.ShapeDtypeStruct((N_LOOKUPS, EMBEDDING_DIM), jnp.float32),
        mesh=mesh,
        scratch_types=[
            pltpu.VMEM((NCHUNK, CHUNK), jnp.int32),
            pltpu.VMEM((NBUF, CHUNK, EMBEDDING_DIM), jnp.float32),
            pltpu.SemaphoreType.DMA((NBUF,)),
            pltpu.SemaphoreType.DMA((NBUF,)),
        ],
        compiler_params=pltpu.CompilerParams(use_tc_tiling_on_sc=False),
    )
    return f(x_w, table)


def kernel(x, table):
    x_flat = x.reshape(-1).astype(jnp.int32)
    x_w = x_flat.reshape(NW, NCHUNK, CHUNK)
    out = _embedding_lookup(x_w, table)
    return out.reshape(x.shape + (EMBEDDING_DIM,))
